# baseline (speedup 1.0000x reference)
import functools

import jax
import jax.numpy as jnp
from jax import lax
from jax.experimental import pallas as pl
from jax.experimental.pallas import tpu as pltpu
from jax.experimental.pallas import tpu_sc as plsc

N_NODES = 50000
ATOM_DIM = 32
BOND_DIM = 16
N_EDGES = 100000

NC = 2
NS = 16
NW = NC * NS

G_CHUNK = 3136
G_TAIL = N_EDGES - (NW - 1) * G_CHUNK

TC_BLOCK = 2048
TC_GRID = (N_EDGES + TC_BLOCK - 1) // TC_BLOCK

S_CHUNK = 3136
S_NCHUNK = 32
S_TAIL = N_EDGES - (S_NCHUNK - 1) * S_CHUNK
ACC_ROWS = 50048
ZERO_ROWS = ACC_ROWS // NS
OUT_ROWS = N_NODES // NS
HALF = ATOM_DIM // 2


def _deinterleave(pair_v, col_v, column, size):
    lanes = jax.lax.broadcasted_iota(jnp.int32, (16,), 0)
    cols = jnp.full((16,), column, jnp.int32)

    def body(i, _):
        rows = i * 16 + lanes
        col_v[pl.ds(i * 16, 16)] = plsc.load_gather(pair_v, [rows, cols])
        return 0

    lax.fori_loop(0, size // 16, body, 0)


def _gather_body(atom_hbm, pair_hbm, out_hbm, pair_v, idx_v, rows_v, sem):
    wid = lax.axis_index("s") * NC + lax.axis_index("c")
    base = wid * G_CHUNK

    def go(size):
        pltpu.sync_copy(pair_hbm.at[pl.ds(base, size)],
                        pair_v.at[pl.ds(0, size)])
        _deinterleave(pair_v, idx_v, 1, size)
        pltpu.async_copy(atom_hbm.at[idx_v.at[pl.ds(0, size)]],
                         rows_v.at[pl.ds(0, size)], sem).wait()
        pltpu.sync_copy(rows_v.at[pl.ds(0, size)],
                        out_hbm.at[pl.ds(base, size)])

    @pl.when(wid < NW - 1)
    def _():
        go(G_CHUNK)

    @pl.when(wid == NW - 1)
    def _():
        go(G_TAIL)


def _sc_gather(atom_features, pair_indices):
    mesh = plsc.VectorSubcoreMesh(core_axis_name="c", subcore_axis_name="s")
    k = functools.partial(
        pl.kernel,
        mesh=mesh,
        out_type=jax.ShapeDtypeStruct((N_EDGES, ATOM_DIM), jnp.float32),
        scratch_types=[
            pltpu.VMEM((G_CHUNK, 2), jnp.int32),
            pltpu.VMEM((G_CHUNK,), jnp.int32),
            pltpu.VMEM((G_CHUNK, ATOM_DIM), jnp.float32),
            pltpu.SemaphoreType.DMA,
        ],
        compiler_params=pltpu.CompilerParams(use_tc_tiling_on_sc=False, needs_layout_passes=False),
    )(_gather_body)
    return k(atom_features, pair_indices)


def _tc_body(bond_ref, nbr_ref, wt2_ref, r_ref, f_ref, b2t_ref,
             tlo_ref, thi_ref):
    bond = bond_ref[...]
    nbr = nbr_ref[...]
    bond_rep = jnp.dot(bond, r_ref[...], preferred_element_type=jnp.float32)
    g = jnp.dot(nbr, wt2_ref[...], preferred_element_type=jnp.float32)
    tr = jnp.dot(bond_rep * g, f_ref[...], preferred_element_type=jnp.float32)
    tr = tr + jnp.dot(nbr, b2t_ref[...], preferred_element_type=jnp.float32)
    tlo_ref[...] = tr[:, :HALF]
    thi_ref[...] = tr[:, HALF:]


def _tc_transform(bond_features, nbr_feats, wt2, r, f, b2t):
    out_shape = [
        jax.ShapeDtypeStruct((N_EDGES, HALF), jnp.float32),
        jax.ShapeDtypeStruct((N_EDGES, HALF), jnp.float32),
    ]
    kdim = BOND_DIM * ATOM_DIM
    return pl.pallas_call(
        _tc_body,
        grid=(TC_GRID,),
        in_specs=[
            pl.BlockSpec((TC_BLOCK, BOND_DIM), lambda i: (i, 0)),
            pl.BlockSpec((TC_BLOCK, ATOM_DIM), lambda i: (i, 0)),
            pl.BlockSpec((ATOM_DIM, kdim), lambda i: (0, 0)),
            pl.BlockSpec((BOND_DIM, kdim), lambda i: (0, 0)),
            pl.BlockSpec((kdim, ATOM_DIM), lambda i: (0, 0)),
            pl.BlockSpec((ATOM_DIM, ATOM_DIM), lambda i: (0, 0)),
        ],
        out_specs=[
            pl.BlockSpec((TC_BLOCK, HALF), lambda i: (i, 0)),
            pl.BlockSpec((TC_BLOCK, HALF), lambda i: (i, 0)),
        ],
        out_shape=out_shape,
    )(bond_features, nbr_feats, wt2, r, f, b2t)


def _scatter_chunk(pair_hbm, t_hbm, acc, pair_v, idx_v, rows_v, base, size):
    pltpu.sync_copy(pair_hbm.at[pl.ds(base, size)], pair_v.at[pl.ds(0, size)])
    _deinterleave(pair_v, idx_v, 0, size)
    pltpu.sync_copy(t_hbm.at[pl.ds(base, size)], rows_v.at[pl.ds(0, size)])
    pltpu.sync_copy(rows_v.at[pl.ds(0, size)],
                    acc.at[idx_v.at[pl.ds(0, size)]], add=True)


def _scatter_body(pair_hbm, tlo_hbm, thi_hbm, zeros_hbm, out_hbm,
                  acc, pair_v, idx_v, rows_v):
    cid = lax.axis_index("c")
    sid = lax.axis_index("s")
    pltpu.sync_copy(zeros_hbm, acc.at[pl.ds(sid * ZERO_ROWS, ZERO_ROWS)])
    plsc.subcore_barrier()

    def do(base, size):
        @pl.when(cid == 0)
        def _():
            _scatter_chunk(pair_hbm, tlo_hbm, acc, pair_v, idx_v, rows_v,
                           base, size)

        @pl.when(cid == 1)
        def _():
            _scatter_chunk(pair_hbm, thi_hbm, acc, pair_v, idx_v, rows_v,
                           base, size)

    do(sid * S_CHUNK, S_CHUNK)

    @pl.when(sid < NS - 1)
    def _():
        do((NS + sid) * S_CHUNK, S_CHUNK)

    @pl.when(sid == NS - 1)
    def _():
        do((S_NCHUNK - 1) * S_CHUNK, S_TAIL)

    plsc.subcore_barrier()
    obase = sid * OUT_ROWS
    pltpu.sync_copy(acc.at[pl.ds(obase, OUT_ROWS)],
                    out_hbm.at[pl.ds(obase, OUT_ROWS), pl.ds(cid * HALF, HALF)])


def _sc_scatter(pair_indices, tlo, thi, zeros_block):
    mesh = plsc.VectorSubcoreMesh(core_axis_name="c", subcore_axis_name="s")
    k = functools.partial(
        pl.kernel,
        mesh=mesh,
        out_type=jax.ShapeDtypeStruct((N_NODES, ATOM_DIM), jnp.float32),
        scratch_types=[
            pltpu.VMEM_SHARED((ACC_ROWS, HALF), jnp.float32),
            pltpu.VMEM((S_CHUNK, 2), jnp.int32),
            pltpu.VMEM((S_CHUNK,), jnp.int32),
            pltpu.VMEM((S_CHUNK, HALF), jnp.float32),
        ],
        compiler_params=pltpu.CompilerParams(use_tc_tiling_on_sc=False, needs_layout_passes=False),
    )(_scatter_body)
    return k(pair_indices, tlo, thi, zeros_block)


def kernel(atom_features, bond_features, pair_indices, kernel, bias):
    kdim = BOND_DIM * ATOM_DIM
    wt2 = kernel.reshape(BOND_DIM, ATOM_DIM, ATOM_DIM).transpose(2, 0, 1)
    wt2 = wt2.reshape(ATOM_DIM, kdim)
    b2t = bias.reshape(ATOM_DIM, ATOM_DIM).T
    c_ids = jnp.arange(kdim, dtype=jnp.int32)
    r = (c_ids[None, :] // ATOM_DIM
         == jnp.arange(BOND_DIM, dtype=jnp.int32)[:, None]).astype(jnp.float32)
    f = (c_ids[:, None] % ATOM_DIM
         == jnp.arange(ATOM_DIM, dtype=jnp.int32)[None, :]).astype(jnp.float32)
    zeros_block = jnp.zeros((ZERO_ROWS, HALF), jnp.float32)

    nbr_feats = _sc_gather(atom_features, pair_indices)
    tlo, thi = _tc_transform(bond_features, nbr_feats, wt2, r, f, b2t)
    return _sc_scatter(pair_indices, tlo, thi, zeros_block)


# speedup vs baseline: 1.2387x; 1.2387x over previous
import functools

import jax
import jax.numpy as jnp
from jax import lax
from jax.experimental import pallas as pl
from jax.experimental.pallas import tpu as pltpu
from jax.experimental.pallas import tpu_sc as plsc

N_NODES = 50000
ATOM_DIM = 32
BOND_DIM = 16
N_EDGES = 100000

NC = 2
NS = 16
NW = NC * NS

G_CHUNK = 3136
G_TAIL = N_EDGES - (NW - 1) * G_CHUNK

TC_BLOCK = 2048
TC_GRID = (N_EDGES + TC_BLOCK - 1) // TC_BLOCK

S_CHUNK = 3136
S_NCHUNK = 32
S_TAIL = N_EDGES - (S_NCHUNK - 1) * S_CHUNK
ACC_ROWS = 50048
ZERO_ROWS = ACC_ROWS // NS
OUT_ROWS = N_NODES // NS
HALF = ATOM_DIM // 2


def _gather_body(atom_hbm, idx_hbm, out_hbm, idx_v, rows_v, sem):
    wid = lax.axis_index("s") * NC + lax.axis_index("c")
    base = wid * G_CHUNK

    def go(size):
        pltpu.sync_copy(idx_hbm.at[pl.ds(base, size)],
                        idx_v.at[pl.ds(0, size)])
        pltpu.async_copy(atom_hbm.at[idx_v.at[pl.ds(0, size)]],
                         rows_v.at[pl.ds(0, size)], sem).wait()
        pltpu.sync_copy(rows_v.at[pl.ds(0, size)],
                        out_hbm.at[pl.ds(base, size)])

    @pl.when(wid < NW - 1)
    def _():
        go(G_CHUNK)

    @pl.when(wid == NW - 1)
    def _():
        go(G_TAIL)


def _sc_gather(atom_features, nbr_idx):
    mesh = plsc.VectorSubcoreMesh(core_axis_name="c", subcore_axis_name="s")
    k = functools.partial(
        pl.kernel,
        mesh=mesh,
        out_type=jax.ShapeDtypeStruct((N_EDGES, ATOM_DIM), jnp.float32),
        scratch_types=[
            pltpu.VMEM((G_CHUNK,), jnp.int32),
            pltpu.VMEM((G_CHUNK, ATOM_DIM), jnp.float32),
            pltpu.SemaphoreType.DMA,
        ],
        compiler_params=pltpu.CompilerParams(use_tc_tiling_on_sc=False),
    )(_gather_body)
    return k(atom_features, nbr_idx)


def _tc_body(bond_ref, nbr_ref, wt2_ref, r_ref, f_ref, b2t_ref,
             tlo_ref, thi_ref):
    bond = bond_ref[...]
    nbr = nbr_ref[...]
    bond_rep = jnp.dot(bond, r_ref[...], preferred_element_type=jnp.float32)
    g = jnp.dot(nbr, wt2_ref[...], preferred_element_type=jnp.float32)
    tr = jnp.dot(bond_rep * g, f_ref[...], preferred_element_type=jnp.float32)
    tr = tr + jnp.dot(nbr, b2t_ref[...], preferred_element_type=jnp.float32)
    tlo_ref[...] = tr[:, :HALF]
    thi_ref[...] = tr[:, HALF:]


def _tc_transform(bond_features, nbr_feats, wt2, r, f, b2t):
    out_shape = [
        jax.ShapeDtypeStruct((N_EDGES, HALF), jnp.float32),
        jax.ShapeDtypeStruct((N_EDGES, HALF), jnp.float32),
    ]
    kdim = BOND_DIM * ATOM_DIM
    return pl.pallas_call(
        _tc_body,
        grid=(TC_GRID,),
        in_specs=[
            pl.BlockSpec((TC_BLOCK, BOND_DIM), lambda i: (i, 0)),
            pl.BlockSpec((TC_BLOCK, ATOM_DIM), lambda i: (i, 0)),
            pl.BlockSpec((ATOM_DIM, kdim), lambda i: (0, 0)),
            pl.BlockSpec((BOND_DIM, kdim), lambda i: (0, 0)),
            pl.BlockSpec((kdim, ATOM_DIM), lambda i: (0, 0)),
            pl.BlockSpec((ATOM_DIM, ATOM_DIM), lambda i: (0, 0)),
        ],
        out_specs=[
            pl.BlockSpec((TC_BLOCK, HALF), lambda i: (i, 0)),
            pl.BlockSpec((TC_BLOCK, HALF), lambda i: (i, 0)),
        ],
        out_shape=out_shape,
    )(bond_features, nbr_feats, wt2, r, f, b2t)


def _scatter_chunk(src_hbm, t_hbm, acc, idx_v, rows_v, base, size):
    pltpu.sync_copy(src_hbm.at[pl.ds(base, size)], idx_v.at[pl.ds(0, size)])
    pltpu.sync_copy(t_hbm.at[pl.ds(base, size)], rows_v.at[pl.ds(0, size)])
    pltpu.sync_copy(rows_v.at[pl.ds(0, size)],
                    acc.at[idx_v.at[pl.ds(0, size)]], add=True)


def _scatter_body(src_hbm, tlo_hbm, thi_hbm, zeros_hbm, out_hbm,
                  acc, idx_v, rows_v):
    cid = lax.axis_index("c")
    sid = lax.axis_index("s")
    pltpu.sync_copy(zeros_hbm, acc.at[pl.ds(sid * ZERO_ROWS, ZERO_ROWS)])
    plsc.subcore_barrier()

    def do(base, size):
        @pl.when(cid == 0)
        def _():
            _scatter_chunk(src_hbm, tlo_hbm, acc, idx_v, rows_v, base, size)

        @pl.when(cid == 1)
        def _():
            _scatter_chunk(src_hbm, thi_hbm, acc, idx_v, rows_v, base, size)

    do(sid * S_CHUNK, S_CHUNK)

    @pl.when(sid < NS - 1)
    def _():
        do((NS + sid) * S_CHUNK, S_CHUNK)

    @pl.when(sid == NS - 1)
    def _():
        do((S_NCHUNK - 1) * S_CHUNK, S_TAIL)

    plsc.subcore_barrier()
    obase = sid * OUT_ROWS
    pltpu.sync_copy(acc.at[pl.ds(obase, OUT_ROWS)],
                    out_hbm.at[pl.ds(obase, OUT_ROWS), pl.ds(cid * HALF, HALF)])


def _sc_scatter(src, tlo, thi, zeros_block):
    mesh = plsc.VectorSubcoreMesh(core_axis_name="c", subcore_axis_name="s")
    k = functools.partial(
        pl.kernel,
        mesh=mesh,
        out_type=jax.ShapeDtypeStruct((N_NODES, ATOM_DIM), jnp.float32),
        scratch_types=[
            pltpu.VMEM_SHARED((ACC_ROWS, HALF), jnp.float32),
            pltpu.VMEM((S_CHUNK,), jnp.int32),
            pltpu.VMEM((S_CHUNK, HALF), jnp.float32),
        ],
        compiler_params=pltpu.CompilerParams(use_tc_tiling_on_sc=False),
    )(_scatter_body)
    return k(src, tlo, thi, zeros_block)


def kernel(atom_features, bond_features, pair_indices, kernel, bias):
    cols = pair_indices.T
    src = cols[0]
    nbr = cols[1]
    kdim = BOND_DIM * ATOM_DIM
    wt2 = kernel.reshape(BOND_DIM, ATOM_DIM, ATOM_DIM).transpose(2, 0, 1)
    wt2 = wt2.reshape(ATOM_DIM, kdim)
    b2t = bias.reshape(ATOM_DIM, ATOM_DIM).T
    c_ids = jnp.arange(kdim, dtype=jnp.int32)
    r = (c_ids[None, :] // ATOM_DIM
         == jnp.arange(BOND_DIM, dtype=jnp.int32)[:, None]).astype(jnp.float32)
    f = (c_ids[:, None] % ATOM_DIM
         == jnp.arange(ATOM_DIM, dtype=jnp.int32)[None, :]).astype(jnp.float32)
    zeros_block = jnp.zeros((ZERO_ROWS, HALF), jnp.float32)

    nbr_feats = _sc_gather(atom_features, nbr)
    tlo, thi = _tc_transform(bond_features, nbr_feats, wt2, r, f, b2t)
    return _sc_scatter(src, tlo, thi, zeros_block)


# speedup vs baseline: 1.7285x; 1.3954x over previous
import functools

import jax
import jax.numpy as jnp
from jax import lax
from jax.experimental import pallas as pl
from jax.experimental.pallas import tpu as pltpu
from jax.experimental.pallas import tpu_sc as plsc

N_NODES = 50000
ATOM_DIM = 32
BOND_DIM = 16
N_EDGES = 100000
QUARTER = N_EDGES // 4
Q_ROWS = QUARTER

NC = 2
NS = 16
NW = NC * NS

W_CHUNK = 3128
W_TAIL = QUARTER - 7 * W_CHUNK

TC_BLOCK4 = 512
TC_GRID = (Q_ROWS + TC_BLOCK4 - 1) // TC_BLOCK4

ACC_ROWS = 50048
ZERO_ROWS = ACC_ROWS // NS
OUT_ROWS = N_NODES // NS
HALF = ATOM_DIM // 2


def _gather_body(atom_hbm, idx_hbm, out_hbm, idx_v, rows_v, sem):
    wid = lax.axis_index("s") * NC + lax.axis_index("c")
    q = wid // 8
    j = wid % 8
    base_e = q * QUARTER + j * W_CHUNK
    base_r = j * W_CHUNK

    def go(size):
        pltpu.sync_copy(idx_hbm.at[pl.ds(base_e, size)],
                        idx_v.at[pl.ds(0, size)])
        pltpu.async_copy(atom_hbm.at[idx_v.at[pl.ds(0, size)]],
                         rows_v.at[pl.ds(0, size)], sem).wait()
        pltpu.sync_copy(rows_v.at[pl.ds(0, size)],
                        out_hbm.at[pl.ds(base_r, size),
                                   pl.ds(q * ATOM_DIM, ATOM_DIM)])

    @pl.when(j < 7)
    def _():
        go(W_CHUNK)

    @pl.when(j == 7)
    def _():
        go(W_TAIL)


def _sc_gather(atom_features, nbr_idx):
    mesh = plsc.VectorSubcoreMesh(core_axis_name="c", subcore_axis_name="s")
    k = functools.partial(
        pl.kernel,
        mesh=mesh,
        out_type=jax.ShapeDtypeStruct((Q_ROWS, 128), jnp.float32),
        scratch_types=[
            pltpu.VMEM((W_CHUNK,), jnp.int32),
            pltpu.VMEM((W_CHUNK, ATOM_DIM), jnp.float32),
            pltpu.SemaphoreType.DMA,
        ],
        compiler_params=pltpu.CompilerParams(use_tc_tiling_on_sc=False),
    )(_gather_body)
    return k(atom_features, nbr_idx)


def _tc_body(bond_ref, nbr_ref, wbig_ref, rbig_ref, fbig_ref, bbig_ref,
             out_ref):
    bond4 = bond_ref[...]
    nbr4 = nbr_ref[...]
    bond_rep = jnp.dot(bond4, rbig_ref[...],
                       preferred_element_type=jnp.float32)
    g = jnp.dot(nbr4, wbig_ref[...], preferred_element_type=jnp.float32)
    tr = jnp.dot(bond_rep * g, fbig_ref[...],
                 preferred_element_type=jnp.float32)
    tr = tr + jnp.dot(nbr4, bbig_ref[...], preferred_element_type=jnp.float32)
    out_ref[...] = tr


def _tc_transform(bond4, nbr4, wbig, rbig, fbig, bbig):
    return pl.pallas_call(
        _tc_body,
        grid=(TC_GRID,),
        in_specs=[
            pl.BlockSpec((TC_BLOCK4, 64), lambda i: (i, 0)),
            pl.BlockSpec((TC_BLOCK4, 128), lambda i: (i, 0)),
            pl.BlockSpec((128, 2048), lambda i: (0, 0)),
            pl.BlockSpec((64, 2048), lambda i: (0, 0)),
            pl.BlockSpec((2048, 128), lambda i: (0, 0)),
            pl.BlockSpec((128, 128), lambda i: (0, 0)),
        ],
        out_specs=pl.BlockSpec((TC_BLOCK4, 128), lambda i: (i, 0)),
        out_shape=jax.ShapeDtypeStruct((Q_ROWS, 128), jnp.float32),
    )(bond4, nbr4, wbig, rbig, fbig, bbig)


def _scatter_body(src_hbm, t_hbm, zeros_hbm, out_hbm, acc, idx_v, rows_v):
    cid = lax.axis_index("c")
    sid = lax.axis_index("s")
    pltpu.sync_copy(zeros_hbm, acc.at[pl.ds(sid * ZERO_ROWS, ZERO_ROWS)])
    plsc.subcore_barrier()

    def do(c, size):
        q = c // 8
        j = c % 8
        base_e = q * QUARTER + j * W_CHUNK
        base_r = j * W_CHUNK
        pltpu.sync_copy(src_hbm.at[pl.ds(base_e, size)],
                        idx_v.at[pl.ds(0, size)])
        pltpu.sync_copy(
            t_hbm.at[pl.ds(base_r, size),
                     pl.ds(q * ATOM_DIM + cid * HALF, HALF)],
            rows_v.at[pl.ds(0, size)])
        pltpu.sync_copy(rows_v.at[pl.ds(0, size)],
                        acc.at[idx_v.at[pl.ds(0, size)]], add=True)

    for step in range(2):
        c = sid + NS * step

        @pl.when(c % 8 < 7)
        def _():
            do(c, W_CHUNK)

        @pl.when(c % 8 == 7)
        def _():
            do(c, W_TAIL)

    plsc.subcore_barrier()
    obase = sid * OUT_ROWS
    pltpu.sync_copy(acc.at[pl.ds(obase, OUT_ROWS)],
                    out_hbm.at[pl.ds(obase, OUT_ROWS), pl.ds(cid * HALF, HALF)])


def _sc_scatter(src, t4, zeros_block):
    mesh = plsc.VectorSubcoreMesh(core_axis_name="c", subcore_axis_name="s")
    k = functools.partial(
        pl.kernel,
        mesh=mesh,
        out_type=jax.ShapeDtypeStruct((N_NODES, ATOM_DIM), jnp.float32),
        scratch_types=[
            pltpu.VMEM_SHARED((ACC_ROWS, HALF), jnp.float32),
            pltpu.VMEM((W_CHUNK,), jnp.int32),
            pltpu.VMEM((W_CHUNK, HALF), jnp.float32),
        ],
        compiler_params=pltpu.CompilerParams(use_tc_tiling_on_sc=False),
    )(_scatter_body)
    return k(src, t4, zeros_block)


def kernel(atom_features, bond_features, pair_indices, kernel, bias):
    cols = pair_indices.T
    src = cols[0]
    nbr = cols[1]
    kdim = BOND_DIM * ATOM_DIM
    wt2 = kernel.reshape(BOND_DIM, ATOM_DIM, ATOM_DIM).transpose(2, 0, 1)
    wt2 = wt2.reshape(ATOM_DIM, kdim)
    b2t = bias.reshape(ATOM_DIM, ATOM_DIM).T
    c_ids = jnp.arange(kdim, dtype=jnp.int32)
    r = (c_ids[None, :] // ATOM_DIM
         == jnp.arange(BOND_DIM, dtype=jnp.int32)[:, None]).astype(jnp.float32)
    f = (c_ids[:, None] % ATOM_DIM
         == jnp.arange(ATOM_DIM, dtype=jnp.int32)[None, :]).astype(jnp.float32)
    eye4 = jnp.eye(4, dtype=jnp.float32)
    wbig = jnp.kron(eye4, wt2)
    rbig = jnp.kron(eye4, r)
    fbig = jnp.kron(eye4, f)
    bbig = jnp.kron(eye4, b2t)
    bond4 = bond_features.reshape(4, QUARTER, BOND_DIM)
    bond4 = bond4.transpose(1, 0, 2).reshape(QUARTER, 4 * BOND_DIM)
    zeros_block = jnp.zeros((ZERO_ROWS, HALF), jnp.float32)

    nbr4 = _sc_gather(atom_features, nbr)
    t4 = _tc_transform(bond4, nbr4, wbig, rbig, fbig, bbig)
    return _sc_scatter(src, t4, zeros_block)


# speedup vs baseline: 1.7668x; 1.0221x over previous
import functools

import jax
import jax.numpy as jnp
from jax import lax
from jax.experimental import pallas as pl
from jax.experimental.pallas import tpu as pltpu
from jax.experimental.pallas import tpu_sc as plsc

N_NODES = 50000
ATOM_DIM = 32
BOND_DIM = 16
N_EDGES = 100000
QUARTER = N_EDGES // 4
Q_ROWS = QUARTER

NC = 2
NS = 16
NW = NC * NS

W_CHUNK = 3128
W_TAIL = QUARTER - 7 * W_CHUNK

TC_BLOCK4 = 1024
TC_GRID = (Q_ROWS + TC_BLOCK4 - 1) // TC_BLOCK4

ACC_ROWS = 50048
ZERO_ROWS = ACC_ROWS // NS
OUT_ROWS = N_NODES // NS
HALF = ATOM_DIM // 2


def _gather_body(atom_hbm, idx_hbm, out_hbm, idx_v, rows_v, sem):
    wid = lax.axis_index("s") * NC + lax.axis_index("c")
    q = wid // 8
    j = wid % 8
    base_e = q * QUARTER + j * W_CHUNK
    base_r = j * W_CHUNK

    def go(size):
        pltpu.sync_copy(idx_hbm.at[pl.ds(base_e, size)],
                        idx_v.at[pl.ds(0, size)])
        pltpu.async_copy(atom_hbm.at[idx_v.at[pl.ds(0, size)]],
                         rows_v.at[pl.ds(0, size)], sem).wait()
        pltpu.sync_copy(rows_v.at[pl.ds(0, size)],
                        out_hbm.at[pl.ds(base_r, size),
                                   pl.ds(q * ATOM_DIM, ATOM_DIM)])

    @pl.when(j < 7)
    def _():
        go(W_CHUNK)

    @pl.when(j == 7)
    def _():
        go(W_TAIL)


def _sc_gather(atom_features, nbr_idx):
    mesh = plsc.VectorSubcoreMesh(core_axis_name="c", subcore_axis_name="s")
    k = functools.partial(
        pl.kernel,
        mesh=mesh,
        out_type=jax.ShapeDtypeStruct((Q_ROWS, 128), jnp.float32),
        scratch_types=[
            pltpu.VMEM((W_CHUNK,), jnp.int32),
            pltpu.VMEM((W_CHUNK, ATOM_DIM), jnp.float32),
            pltpu.SemaphoreType.DMA,
        ],
        compiler_params=pltpu.CompilerParams(use_tc_tiling_on_sc=False),
    )(_gather_body)
    return k(atom_features, nbr_idx)


def _tc_body(bond_ref, nbr_ref, wbig_ref, rbig_ref, fbig_ref, bbig_ref,
             out_ref):
    bond4 = bond_ref[...]
    nbr4 = nbr_ref[...]
    bond_rep = jnp.dot(bond4, rbig_ref[...],
                       preferred_element_type=jnp.float32)
    g = jnp.dot(nbr4, wbig_ref[...], preferred_element_type=jnp.float32)
    tr = jnp.dot(bond_rep * g, fbig_ref[...],
                 preferred_element_type=jnp.float32)
    tr = tr + jnp.dot(nbr4, bbig_ref[...], preferred_element_type=jnp.float32)
    out_ref[...] = tr


def _tc_transform(bond4, nbr4, wbig, rbig, fbig, bbig):
    return pl.pallas_call(
        _tc_body,
        grid=(TC_GRID,),
        in_specs=[
            pl.BlockSpec((TC_BLOCK4, 64), lambda i: (i, 0)),
            pl.BlockSpec((TC_BLOCK4, 128), lambda i: (i, 0)),
            pl.BlockSpec((128, 2048), lambda i: (0, 0)),
            pl.BlockSpec((64, 2048), lambda i: (0, 0)),
            pl.BlockSpec((2048, 128), lambda i: (0, 0)),
            pl.BlockSpec((128, 128), lambda i: (0, 0)),
        ],
        out_specs=pl.BlockSpec((TC_BLOCK4, 128), lambda i: (i, 0)),
        out_shape=jax.ShapeDtypeStruct((Q_ROWS, 128), jnp.float32),
    )(bond4, nbr4, wbig, rbig, fbig, bbig)


def _scatter_body(src_hbm, t_hbm, zeros_hbm, out_hbm, acc, idx_v, rows_v):
    cid = lax.axis_index("c")
    sid = lax.axis_index("s")
    pltpu.sync_copy(zeros_hbm, acc.at[pl.ds(sid * ZERO_ROWS, ZERO_ROWS)])
    plsc.subcore_barrier()

    def do(c, size):
        q = c // 8
        j = c % 8
        base_e = q * QUARTER + j * W_CHUNK
        base_r = j * W_CHUNK
        pltpu.sync_copy(src_hbm.at[pl.ds(base_e, size)],
                        idx_v.at[pl.ds(0, size)])
        pltpu.sync_copy(
            t_hbm.at[pl.ds(base_r, size),
                     pl.ds(q * ATOM_DIM + cid * HALF, HALF)],
            rows_v.at[pl.ds(0, size)])
        pltpu.sync_copy(rows_v.at[pl.ds(0, size)],
                        acc.at[idx_v.at[pl.ds(0, size)]], add=True)

    for step in range(2):
        c = sid + NS * step

        @pl.when(c % 8 < 7)
        def _():
            do(c, W_CHUNK)

        @pl.when(c % 8 == 7)
        def _():
            do(c, W_TAIL)

    plsc.subcore_barrier()
    obase = sid * OUT_ROWS
    pltpu.sync_copy(acc.at[pl.ds(obase, OUT_ROWS)],
                    out_hbm.at[pl.ds(obase, OUT_ROWS), pl.ds(cid * HALF, HALF)])


def _sc_scatter(src, t4, zeros_block):
    mesh = plsc.VectorSubcoreMesh(core_axis_name="c", subcore_axis_name="s")
    k = functools.partial(
        pl.kernel,
        mesh=mesh,
        out_type=jax.ShapeDtypeStruct((N_NODES, ATOM_DIM), jnp.float32),
        scratch_types=[
            pltpu.VMEM_SHARED((ACC_ROWS, HALF), jnp.float32),
            pltpu.VMEM((W_CHUNK,), jnp.int32),
            pltpu.VMEM((W_CHUNK, HALF), jnp.float32),
        ],
        compiler_params=pltpu.CompilerParams(use_tc_tiling_on_sc=False),
    )(_scatter_body)
    return k(src, t4, zeros_block)


def kernel(atom_features, bond_features, pair_indices, kernel, bias):
    cols = pair_indices.T
    src = cols[0]
    nbr = cols[1]
    kdim = BOND_DIM * ATOM_DIM
    wt2 = kernel.reshape(BOND_DIM, ATOM_DIM, ATOM_DIM).transpose(2, 0, 1)
    wt2 = wt2.reshape(ATOM_DIM, kdim)
    b2t = bias.reshape(ATOM_DIM, ATOM_DIM).T
    c_ids = jnp.arange(kdim, dtype=jnp.int32)
    r = (c_ids[None, :] // ATOM_DIM
         == jnp.arange(BOND_DIM, dtype=jnp.int32)[:, None]).astype(jnp.float32)
    f = (c_ids[:, None] % ATOM_DIM
         == jnp.arange(ATOM_DIM, dtype=jnp.int32)[None, :]).astype(jnp.float32)
    eye4 = jnp.eye(4, dtype=jnp.float32)
    wbig = jnp.kron(eye4, wt2)
    rbig = jnp.kron(eye4, r)
    fbig = jnp.kron(eye4, f)
    bbig = jnp.kron(eye4, b2t)
    bond4 = bond_features.reshape(4, QUARTER, BOND_DIM)
    bond4 = bond4.transpose(1, 0, 2).reshape(QUARTER, 4 * BOND_DIM)
    zeros_block = jnp.zeros((ZERO_ROWS, HALF), jnp.float32)

    nbr4 = _sc_gather(atom_features, nbr)
    t4 = _tc_transform(bond4, nbr4, wbig, rbig, fbig, bbig)
    return _sc_scatter(src, t4, zeros_block)


# speedup vs baseline: 1.7772x; 1.0059x over previous
import functools

import jax
import jax.numpy as jnp
from jax import lax
from jax.experimental import pallas as pl
from jax.experimental.pallas import tpu as pltpu
from jax.experimental.pallas import tpu_sc as plsc

N_NODES = 50000
ATOM_DIM = 32
BOND_DIM = 16
N_EDGES = 100000
QUARTER = N_EDGES // 4
Q_ROWS = QUARTER

NC = 2
NS = 16
NW = NC * NS

W_CHUNK = 3128
W_TAIL = QUARTER - 7 * W_CHUNK

TC_BLOCK4 = 1024
TC_GRID = (Q_ROWS + TC_BLOCK4 - 1) // TC_BLOCK4

ACC_ROWS = 50048
ZERO_ROWS = ACC_ROWS // NS
OUT_ROWS = N_NODES // NS
HALF = ATOM_DIM // 2


def _gather_body(atom_hbm, cols_hbm, out_hbm, idx_v, rows_v, sem):
    wid = lax.axis_index("s") * NC + lax.axis_index("c")
    q = wid // 8
    j = wid % 8
    base_e = q * QUARTER + j * W_CHUNK
    base_r = j * W_CHUNK

    def go(size):
        pltpu.sync_copy(cols_hbm.at[pl.ds(N_EDGES + base_e, size)],
                        idx_v.at[pl.ds(0, size)])
        pltpu.async_copy(atom_hbm.at[idx_v.at[pl.ds(0, size)]],
                         rows_v.at[pl.ds(0, size)], sem).wait()
        pltpu.sync_copy(rows_v.at[pl.ds(0, size)],
                        out_hbm.at[pl.ds(base_r, size),
                                   pl.ds(q * ATOM_DIM, ATOM_DIM)])

    @pl.when(j < 7)
    def _():
        go(W_CHUNK)

    @pl.when(j == 7)
    def _():
        go(W_TAIL)


def _sc_gather(atom_features, cols_flat):
    mesh = plsc.VectorSubcoreMesh(core_axis_name="c", subcore_axis_name="s")
    k = functools.partial(
        pl.kernel,
        mesh=mesh,
        out_type=jax.ShapeDtypeStruct((Q_ROWS, 128), jnp.float32),
        scratch_types=[
            pltpu.VMEM((W_CHUNK,), jnp.int32),
            pltpu.VMEM((W_CHUNK, ATOM_DIM), jnp.float32),
            pltpu.SemaphoreType.DMA,
        ],
        compiler_params=pltpu.CompilerParams(use_tc_tiling_on_sc=False),
    )(_gather_body)
    return k(atom_features, cols_flat)


def _tc_body(bond_ref, nbr_ref, wbig_ref, rbig_ref, fbig_ref, bbig_ref,
             out_ref):
    bond4 = bond_ref[...]
    nbr4 = nbr_ref[...]
    bond_rep = jnp.dot(bond4, rbig_ref[...],
                       preferred_element_type=jnp.float32)
    g = jnp.dot(nbr4, wbig_ref[...], preferred_element_type=jnp.float32)
    tr = jnp.dot(bond_rep * g, fbig_ref[...],
                 preferred_element_type=jnp.float32)
    tr = tr + jnp.dot(nbr4, bbig_ref[...], preferred_element_type=jnp.float32)
    out_ref[...] = tr


def _tc_transform(bond4, nbr4, wbig, rbig, fbig, bbig):
    return pl.pallas_call(
        _tc_body,
        grid=(TC_GRID,),
        in_specs=[
            pl.BlockSpec((TC_BLOCK4, 64), lambda i: (i, 0)),
            pl.BlockSpec((TC_BLOCK4, 128), lambda i: (i, 0)),
            pl.BlockSpec((128, 2048), lambda i: (0, 0)),
            pl.BlockSpec((64, 2048), lambda i: (0, 0)),
            pl.BlockSpec((2048, 128), lambda i: (0, 0)),
            pl.BlockSpec((128, 128), lambda i: (0, 0)),
        ],
        out_specs=pl.BlockSpec((TC_BLOCK4, 128), lambda i: (i, 0)),
        out_shape=jax.ShapeDtypeStruct((Q_ROWS, 128), jnp.float32),
    )(bond4, nbr4, wbig, rbig, fbig, bbig)


def _scatter_body(cols_hbm, t_hbm, zeros_hbm, out_hbm, acc, idx_v, rows_v):
    cid = lax.axis_index("c")
    sid = lax.axis_index("s")
    pltpu.sync_copy(zeros_hbm, acc.at[pl.ds(sid * ZERO_ROWS, ZERO_ROWS)])
    plsc.subcore_barrier()

    def do(c, size):
        q = c // 8
        j = c % 8
        base_e = q * QUARTER + j * W_CHUNK
        base_r = j * W_CHUNK
        pltpu.sync_copy(cols_hbm.at[pl.ds(base_e, size)],
                        idx_v.at[pl.ds(0, size)])
        pltpu.sync_copy(
            t_hbm.at[pl.ds(base_r, size),
                     pl.ds(q * ATOM_DIM + cid * HALF, HALF)],
            rows_v.at[pl.ds(0, size)])
        pltpu.sync_copy(rows_v.at[pl.ds(0, size)],
                        acc.at[idx_v.at[pl.ds(0, size)]], add=True)

    for step in range(2):
        c = sid + NS * step

        @pl.when(c % 8 < 7)
        def _():
            do(c, W_CHUNK)

        @pl.when(c % 8 == 7)
        def _():
            do(c, W_TAIL)

    plsc.subcore_barrier()
    obase = sid * OUT_ROWS
    pltpu.sync_copy(acc.at[pl.ds(obase, OUT_ROWS)],
                    out_hbm.at[pl.ds(obase, OUT_ROWS), pl.ds(cid * HALF, HALF)])


def _sc_scatter(cols_flat, t4, zeros_block):
    mesh = plsc.VectorSubcoreMesh(core_axis_name="c", subcore_axis_name="s")
    k = functools.partial(
        pl.kernel,
        mesh=mesh,
        out_type=jax.ShapeDtypeStruct((N_NODES, ATOM_DIM), jnp.float32),
        scratch_types=[
            pltpu.VMEM_SHARED((ACC_ROWS, HALF), jnp.float32),
            pltpu.VMEM((W_CHUNK,), jnp.int32),
            pltpu.VMEM((W_CHUNK, HALF), jnp.float32),
        ],
        compiler_params=pltpu.CompilerParams(use_tc_tiling_on_sc=False),
    )(_scatter_body)
    return k(cols_flat, t4, zeros_block)


def kernel(atom_features, bond_features, pair_indices, kernel, bias):
    cols_flat = pair_indices.T.reshape(2 * N_EDGES)
    kdim = BOND_DIM * ATOM_DIM
    wt2 = kernel.reshape(BOND_DIM, ATOM_DIM, ATOM_DIM).transpose(2, 0, 1)
    wt2 = wt2.reshape(ATOM_DIM, kdim)
    b2t = bias.reshape(ATOM_DIM, ATOM_DIM).T
    c_ids = jnp.arange(kdim, dtype=jnp.int32)
    r = (c_ids[None, :] // ATOM_DIM
         == jnp.arange(BOND_DIM, dtype=jnp.int32)[:, None]).astype(jnp.float32)
    f = (c_ids[:, None] % ATOM_DIM
         == jnp.arange(ATOM_DIM, dtype=jnp.int32)[None, :]).astype(jnp.float32)
    eye4 = jnp.eye(4, dtype=jnp.float32)
    wbig = jnp.kron(eye4, wt2)
    rbig = jnp.kron(eye4, r)
    fbig = jnp.kron(eye4, f)
    bbig = jnp.kron(eye4, b2t)
    bond4 = bond_features.reshape(4, QUARTER, BOND_DIM)
    bond4 = bond4.transpose(1, 0, 2).reshape(QUARTER, 4 * BOND_DIM)
    zeros_block = jnp.zeros((ZERO_ROWS, HALF), jnp.float32)

    nbr4 = _sc_gather(atom_features, cols_flat)
    t4 = _tc_transform(bond4, nbr4, wbig, rbig, fbig, bbig)
    return _sc_scatter(cols_flat, t4, zeros_block)


# speedup vs baseline: 1.8162x; 1.0219x over previous
import functools

import jax
import jax.numpy as jnp
from jax import lax
from jax.experimental import pallas as pl
from jax.experimental.pallas import tpu as pltpu
from jax.experimental.pallas import tpu_sc as plsc

N_NODES = 50000
ATOM_DIM = 32
BOND_DIM = 16
N_EDGES = 100000
QUARTER = N_EDGES // 4
Q_ROWS = QUARTER

NC = 2
NS = 16
NW = NC * NS

W_CHUNK = 3128
W_TAIL = QUARTER - 7 * W_CHUNK

TC_BLOCK4 = 1024
TC_GRID = (Q_ROWS + TC_BLOCK4 - 1) // TC_BLOCK4

ACC_ROWS = 50048
ZERO_ROWS = ACC_ROWS // NS
OUT_ROWS = N_NODES // NS
HALF = ATOM_DIM // 2


def _gather_body(atom_hbm, cols_hbm, out_hbm, idx_v, rows_v, sem):
    wid = lax.axis_index("s") * NC + lax.axis_index("c")
    q = wid // 8
    j = wid % 8
    base_e = q * QUARTER + j * W_CHUNK
    base_r = j * W_CHUNK

    def go(size):
        pltpu.sync_copy(cols_hbm.at[pl.ds(N_EDGES + base_e, size)],
                        idx_v.at[pl.ds(0, size)])
        pltpu.async_copy(atom_hbm.at[idx_v.at[pl.ds(0, size)]],
                         rows_v.at[pl.ds(0, size)], sem).wait()
        pltpu.sync_copy(rows_v.at[pl.ds(0, size)],
                        out_hbm.at[pl.ds(base_r, size),
                                   pl.ds(q * ATOM_DIM, ATOM_DIM)])

    @pl.when(j < 7)
    def _():
        go(W_CHUNK)

    @pl.when(j == 7)
    def _():
        go(W_TAIL)


def _sc_gather(atom_features, cols_flat):
    mesh = plsc.VectorSubcoreMesh(core_axis_name="c", subcore_axis_name="s")
    k = functools.partial(
        pl.kernel,
        mesh=mesh,
        out_type=jax.ShapeDtypeStruct((Q_ROWS, 128), jnp.float32),
        scratch_types=[
            pltpu.VMEM((W_CHUNK,), jnp.int32),
            pltpu.VMEM((W_CHUNK, ATOM_DIM), jnp.float32),
            pltpu.SemaphoreType.DMA,
        ],
        compiler_params=pltpu.CompilerParams(use_tc_tiling_on_sc=False),
    )(_gather_body)
    return k(atom_features, cols_flat)


def _tc_body(bond_ref, nbr_ref, wbig_ref, rbig_ref, fbig_ref, bbig_ref,
             out_ref):
    bond4 = bond_ref[...]
    nbr4 = nbr_ref[...]
    bond_rep = jnp.dot(bond4, rbig_ref[...],
                       preferred_element_type=jnp.float32)
    g = jnp.dot(nbr4, wbig_ref[...], preferred_element_type=jnp.float32)
    tr = jnp.dot(bond_rep * g, fbig_ref[...],
                 preferred_element_type=jnp.float32)
    tr = tr + jnp.dot(nbr4, bbig_ref[...], preferred_element_type=jnp.float32)
    out_ref[...] = tr


def _tc_transform(bond4, nbr4, wbig, rbig, fbig, bbig):
    return pl.pallas_call(
        _tc_body,
        grid=(TC_GRID,),
        in_specs=[
            pl.BlockSpec((TC_BLOCK4, 64), lambda i: (i, 0)),
            pl.BlockSpec((TC_BLOCK4, 128), lambda i: (i, 0)),
            pl.BlockSpec((128, 2048), lambda i: (0, 0)),
            pl.BlockSpec((64, 2048), lambda i: (0, 0)),
            pl.BlockSpec((2048, 128), lambda i: (0, 0)),
            pl.BlockSpec((128, 128), lambda i: (0, 0)),
        ],
        out_specs=pl.BlockSpec((TC_BLOCK4, 128), lambda i: (i, 0)),
        out_shape=jax.ShapeDtypeStruct((Q_ROWS, 128), jnp.float32),
    )(bond4, nbr4, wbig, rbig, fbig, bbig)


SUB_A = 1568
SUB_B_FULL = W_CHUNK - SUB_A
SUB_B_TAIL = W_TAIL - SUB_A


def _scatter_body(cols_hbm, t_hbm, zeros_hbm, out_hbm, acc,
                  idx_a, idx_b, rows_a, rows_b, sem_ia, sem_ib, sem_ra,
                  sem_rb):
    cid = lax.axis_index("c")
    sid = lax.axis_index("s")
    j = sid % 8
    lane0 = cid * HALF

    bufs = [(idx_a, rows_a, sem_ia, sem_ra), (idx_b, rows_b, sem_ib, sem_rb)]

    def go(sub_b):
        subs = []
        for step in range(2):
            c = sid + NS * step
            q = c // 8
            base_e = q * QUARTER + j * W_CHUNK
            base_r = j * W_CHUNK
            qlane = q * ATOM_DIM + lane0
            subs.append((base_e, base_r, qlane, SUB_A))
            subs.append((base_e + SUB_A, base_r + SUB_A, qlane, sub_b))

        def fetch(i, b):
            base_e, base_r, qlane, size = subs[i]
            idx_v, rows_v, sem_i, sem_r = bufs[b]
            di = pltpu.async_copy(cols_hbm.at[pl.ds(base_e, size)],
                                  idx_v.at[pl.ds(0, size)], sem_i)
            dr = pltpu.async_copy(
                t_hbm.at[pl.ds(base_r, size), pl.ds(qlane, HALF)],
                rows_v.at[pl.ds(0, size)], sem_r)
            return di, dr

        pending = fetch(0, 0)
        pltpu.sync_copy(zeros_hbm, acc.at[pl.ds(sid * ZERO_ROWS, ZERO_ROWS)])
        plsc.subcore_barrier()
        for i in range(4):
            nxt = fetch(i + 1, (i + 1) % 2) if i < 3 else None
            di, dr = pending
            di.wait()
            dr.wait()
            size = subs[i][3]
            idx_v, rows_v, _, _ = bufs[i % 2]
            pltpu.sync_copy(rows_v.at[pl.ds(0, size)],
                            acc.at[idx_v.at[pl.ds(0, size)]], add=True)
            pending = nxt

    @pl.when(j < 7)
    def _():
        go(SUB_B_FULL)

    @pl.when(j == 7)
    def _():
        go(SUB_B_TAIL)

    plsc.subcore_barrier()
    obase = sid * OUT_ROWS
    pltpu.sync_copy(acc.at[pl.ds(obase, OUT_ROWS)],
                    out_hbm.at[pl.ds(obase, OUT_ROWS), pl.ds(cid * HALF, HALF)])


def _sc_scatter(cols_flat, t4, zeros_block):
    mesh = plsc.VectorSubcoreMesh(core_axis_name="c", subcore_axis_name="s")
    k = functools.partial(
        pl.kernel,
        mesh=mesh,
        out_type=jax.ShapeDtypeStruct((N_NODES, ATOM_DIM), jnp.float32),
        scratch_types=[
            pltpu.VMEM_SHARED((ACC_ROWS, HALF), jnp.float32),
            pltpu.VMEM((SUB_A,), jnp.int32),
            pltpu.VMEM((SUB_A,), jnp.int32),
            pltpu.VMEM((SUB_A, HALF), jnp.float32),
            pltpu.VMEM((SUB_A, HALF), jnp.float32),
            pltpu.SemaphoreType.DMA,
            pltpu.SemaphoreType.DMA,
            pltpu.SemaphoreType.DMA,
            pltpu.SemaphoreType.DMA,
        ],
        compiler_params=pltpu.CompilerParams(use_tc_tiling_on_sc=False),
    )(_scatter_body)
    return k(cols_flat, t4, zeros_block)


def kernel(atom_features, bond_features, pair_indices, kernel, bias):
    cols_flat = pair_indices.T.reshape(2 * N_EDGES)
    kdim = BOND_DIM * ATOM_DIM
    wt2 = kernel.reshape(BOND_DIM, ATOM_DIM, ATOM_DIM).transpose(2, 0, 1)
    wt2 = wt2.reshape(ATOM_DIM, kdim)
    b2t = bias.reshape(ATOM_DIM, ATOM_DIM).T
    c_ids = jnp.arange(kdim, dtype=jnp.int32)
    r = (c_ids[None, :] // ATOM_DIM
         == jnp.arange(BOND_DIM, dtype=jnp.int32)[:, None]).astype(jnp.float32)
    f = (c_ids[:, None] % ATOM_DIM
         == jnp.arange(ATOM_DIM, dtype=jnp.int32)[None, :]).astype(jnp.float32)
    eye4 = jnp.eye(4, dtype=jnp.float32)
    wbig = jnp.kron(eye4, wt2)
    rbig = jnp.kron(eye4, r)
    fbig = jnp.kron(eye4, f)
    bbig = jnp.kron(eye4, b2t)
    bond4 = bond_features.reshape(4, QUARTER, BOND_DIM)
    bond4 = bond4.transpose(1, 0, 2).reshape(QUARTER, 4 * BOND_DIM)
    zeros_block = jnp.zeros((ZERO_ROWS, HALF), jnp.float32)

    nbr4 = _sc_gather(atom_features, cols_flat)
    t4 = _tc_transform(bond4, nbr4, wbig, rbig, fbig, bbig)
    return _sc_scatter(cols_flat, t4, zeros_block)


# speedup vs baseline: 1.8318x; 1.0086x over previous
import functools

import jax
import jax.numpy as jnp
from jax import lax
from jax.experimental import pallas as pl
from jax.experimental.pallas import tpu as pltpu
from jax.experimental.pallas import tpu_sc as plsc

N_NODES = 50000
ATOM_DIM = 32
BOND_DIM = 16
N_EDGES = 100000
QUARTER = N_EDGES // 4
Q_ROWS = QUARTER

NC = 2
NS = 16
NW = NC * NS

W_CHUNK = 3128
W_TAIL = QUARTER - 7 * W_CHUNK

SUB_A = 1568
SUB_B_FULL = W_CHUNK - SUB_A
SUB_B_TAIL = W_TAIL - SUB_A

TC_BLOCK4 = 1024
TC_GRID = (Q_ROWS + TC_BLOCK4 - 1) // TC_BLOCK4

ACC_ROWS = 50048
ZERO_ROWS = ACC_ROWS // NS
OUT_ROWS = N_NODES // NS
HALF = ATOM_DIM // 2


def _gather_body(atom_hbm, cols_hbm, out_hbm, idx_a, idx_b, rows_a, rows_b,
                 sem_a, sem_b, sem_g, sem_w):
    wid = lax.axis_index("s") * NC + lax.axis_index("c")
    q = wid // 8
    j = wid % 8
    base_e = q * QUARTER + j * W_CHUNK
    base_r = j * W_CHUNK
    bufs = [(idx_a, rows_a, sem_a), (idx_b, rows_b, sem_b)]

    def go(sub_b):
        sizes = (SUB_A, sub_b)
        offs = (0, SUB_A)
        d0 = pltpu.async_copy(cols_hbm.at[pl.ds(N_EDGES + base_e, SUB_A)],
                              idx_a.at[pl.ds(0, SUB_A)], sem_a)
        d1 = pltpu.async_copy(
            cols_hbm.at[pl.ds(N_EDGES + base_e + SUB_A, sub_b)],
            idx_b.at[pl.ds(0, sub_b)], sem_b)
        gathers = []
        for i in range(2):
            idx_v, rows_v, _ = bufs[i]
            (d0 if i == 0 else d1).wait()
            gathers.append(pltpu.async_copy(
                atom_hbm.at[idx_v.at[pl.ds(0, sizes[i])]],
                rows_v.at[pl.ds(0, sizes[i])], sem_g))
        for i in range(2):
            gathers[i].wait()
            _, rows_v, _ = bufs[i]
            pltpu.async_copy(
                rows_v.at[pl.ds(0, sizes[i])],
                out_hbm.at[pl.ds(base_r + offs[i], sizes[i]),
                           pl.ds(q * ATOM_DIM, ATOM_DIM)], sem_w).wait()

    @pl.when(j < 7)
    def _():
        go(SUB_B_FULL)

    @pl.when(j == 7)
    def _():
        go(SUB_B_TAIL)


def _sc_gather(atom_features, cols_flat):
    mesh = plsc.VectorSubcoreMesh(core_axis_name="c", subcore_axis_name="s")
    k = functools.partial(
        pl.kernel,
        mesh=mesh,
        out_type=jax.ShapeDtypeStruct((Q_ROWS, 128), jnp.float32),
        scratch_types=[
            pltpu.VMEM((SUB_A,), jnp.int32),
            pltpu.VMEM((SUB_A,), jnp.int32),
            pltpu.VMEM((SUB_A, ATOM_DIM), jnp.float32),
            pltpu.VMEM((SUB_A, ATOM_DIM), jnp.float32),
            pltpu.SemaphoreType.DMA,
            pltpu.SemaphoreType.DMA,
            pltpu.SemaphoreType.DMA,
            pltpu.SemaphoreType.DMA,
        ],
        compiler_params=pltpu.CompilerParams(use_tc_tiling_on_sc=False),
    )(_gather_body)
    return k(atom_features, cols_flat)


def _tc_body(bond_ref, nbr_ref, wbig_ref, rbig_ref, fbig_ref, bbig_ref,
             out_ref):
    bond4 = bond_ref[...]
    nbr4 = nbr_ref[...]
    bond_rep = jnp.dot(bond4, rbig_ref[...],
                       preferred_element_type=jnp.float32)
    g = jnp.dot(nbr4, wbig_ref[...], preferred_element_type=jnp.float32)
    tr = jnp.dot(bond_rep * g, fbig_ref[...],
                 preferred_element_type=jnp.float32)
    tr = tr + jnp.dot(nbr4, bbig_ref[...], preferred_element_type=jnp.float32)
    out_ref[...] = tr


def _tc_transform(bond4, nbr4, wbig, rbig, fbig, bbig):
    return pl.pallas_call(
        _tc_body,
        grid=(TC_GRID,),
        in_specs=[
            pl.BlockSpec((TC_BLOCK4, 64), lambda i: (i, 0)),
            pl.BlockSpec((TC_BLOCK4, 128), lambda i: (i, 0)),
            pl.BlockSpec((128, 2048), lambda i: (0, 0)),
            pl.BlockSpec((64, 2048), lambda i: (0, 0)),
            pl.BlockSpec((2048, 128), lambda i: (0, 0)),
            pl.BlockSpec((128, 128), lambda i: (0, 0)),
        ],
        out_specs=pl.BlockSpec((TC_BLOCK4, 128), lambda i: (i, 0)),
        out_shape=jax.ShapeDtypeStruct((Q_ROWS, 128), jnp.float32),
    )(bond4, nbr4, wbig, rbig, fbig, bbig)


def _scatter_body(cols_hbm, t_hbm, zeros_hbm, out_hbm, acc,
                  idx_a, idx_b, rows_a, rows_b, sem_ia, sem_ib, sem_ra,
                  sem_rb):
    cid = lax.axis_index("c")
    sid = lax.axis_index("s")
    j = sid % 8
    lane0 = cid * HALF

    bufs = [(idx_a, rows_a, sem_ia, sem_ra), (idx_b, rows_b, sem_ib, sem_rb)]

    def go(sub_b):
        subs = []
        for step in range(2):
            c = sid + NS * step
            q = c // 8
            base_e = q * QUARTER + j * W_CHUNK
            base_r = j * W_CHUNK
            qlane = q * ATOM_DIM + lane0
            subs.append((base_e, base_r, qlane, SUB_A))
            subs.append((base_e + SUB_A, base_r + SUB_A, qlane, sub_b))

        def fetch(i, b):
            base_e, base_r, qlane, size = subs[i]
            idx_v, rows_v, sem_i, sem_r = bufs[b]
            di = pltpu.async_copy(cols_hbm.at[pl.ds(base_e, size)],
                                  idx_v.at[pl.ds(0, size)], sem_i)
            dr = pltpu.async_copy(
                t_hbm.at[pl.ds(base_r, size), pl.ds(qlane, HALF)],
                rows_v.at[pl.ds(0, size)], sem_r)
            return di, dr

        pending = fetch(0, 0)
        pltpu.sync_copy(zeros_hbm, acc.at[pl.ds(sid * ZERO_ROWS, ZERO_ROWS)])
        plsc.subcore_barrier()
        for i in range(4):
            nxt = fetch(i + 1, (i + 1) % 2) if i < 3 else None
            di, dr = pending
            di.wait()
            dr.wait()
            size = subs[i][3]
            idx_v, rows_v, _, _ = bufs[i % 2]
            pltpu.sync_copy(rows_v.at[pl.ds(0, size)],
                            acc.at[idx_v.at[pl.ds(0, size)]], add=True)
            pending = nxt

    @pl.when(j < 7)
    def _():
        go(SUB_B_FULL)

    @pl.when(j == 7)
    def _():
        go(SUB_B_TAIL)

    plsc.subcore_barrier()
    obase = sid * OUT_ROWS
    pltpu.sync_copy(acc.at[pl.ds(obase, OUT_ROWS)],
                    out_hbm.at[pl.ds(obase, OUT_ROWS), pl.ds(cid * HALF, HALF)])


def _sc_scatter(cols_flat, t4, zeros_block):
    mesh = plsc.VectorSubcoreMesh(core_axis_name="c", subcore_axis_name="s")
    k = functools.partial(
        pl.kernel,
        mesh=mesh,
        out_type=jax.ShapeDtypeStruct((N_NODES, ATOM_DIM), jnp.float32),
        scratch_types=[
            pltpu.VMEM_SHARED((ACC_ROWS, HALF), jnp.float32),
            pltpu.VMEM((SUB_A,), jnp.int32),
            pltpu.VMEM((SUB_A,), jnp.int32),
            pltpu.VMEM((SUB_A, HALF), jnp.float32),
            pltpu.VMEM((SUB_A, HALF), jnp.float32),
            pltpu.SemaphoreType.DMA,
            pltpu.SemaphoreType.DMA,
            pltpu.SemaphoreType.DMA,
            pltpu.SemaphoreType.DMA,
        ],
        compiler_params=pltpu.CompilerParams(use_tc_tiling_on_sc=False),
    )(_scatter_body)
    return k(cols_flat, t4, zeros_block)


def kernel(atom_features, bond_features, pair_indices, kernel, bias):
    cols_flat = pair_indices.T.reshape(2 * N_EDGES)
    kdim = BOND_DIM * ATOM_DIM
    wt2 = kernel.reshape(BOND_DIM, ATOM_DIM, ATOM_DIM).transpose(2, 0, 1)
    wt2 = wt2.reshape(ATOM_DIM, kdim)
    b2t = bias.reshape(ATOM_DIM, ATOM_DIM).T
    c_ids = jnp.arange(kdim, dtype=jnp.int32)
    r = (c_ids[None, :] // ATOM_DIM
         == jnp.arange(BOND_DIM, dtype=jnp.int32)[:, None]).astype(jnp.float32)
    f = (c_ids[:, None] % ATOM_DIM
         == jnp.arange(ATOM_DIM, dtype=jnp.int32)[None, :]).astype(jnp.float32)
    eye4 = jnp.eye(4, dtype=jnp.float32)
    wbig = jnp.kron(eye4, wt2)
    rbig = jnp.kron(eye4, r)
    fbig = jnp.kron(eye4, f)
    bbig = jnp.kron(eye4, b2t)
    bond4 = bond_features.reshape(4, QUARTER, BOND_DIM)
    bond4 = bond4.transpose(1, 0, 2).reshape(QUARTER, 4 * BOND_DIM)
    zeros_block = jnp.zeros((ZERO_ROWS, HALF), jnp.float32)

    nbr4 = _sc_gather(atom_features, cols_flat)
    t4 = _tc_transform(bond4, nbr4, wbig, rbig, fbig, bbig)
    return _sc_scatter(cols_flat, t4, zeros_block)
